# Initial kernel scaffold; baseline (speedup 1.0000x reference)
import jax
import jax.numpy as jnp
from jax.experimental import pallas as pl


def kernel(input, inv_indices):
    raise NotImplementedError("write your pallas kernel here")


# speedup vs baseline: 14.1712x; 14.1712x over previous
import jax
import jax.numpy as jnp
from jax.experimental import pallas as pl

IMW = 224
ROWS = 8 * 192 * 224
BLOCK_ROWS = 2048


def _flip_body(x_ref, p_ref, o_ref):
    o_ref[...] = jnp.dot(x_ref[...], p_ref[...],
                         preferred_element_type=jnp.float32)


def kernel(input, inv_indices):
    x2 = input.reshape(ROWS, IMW)
    perm = (inv_indices[None, :].astype(jnp.int32)
            == jnp.arange(IMW, dtype=jnp.int32)[:, None]).astype(jnp.float32)
    out = pl.pallas_call(
        _flip_body,
        grid=(ROWS // BLOCK_ROWS,),
        in_specs=[
            pl.BlockSpec((BLOCK_ROWS, IMW), lambda i: (i, 0)),
            pl.BlockSpec((IMW, IMW), lambda i: (0, 0)),
        ],
        out_specs=pl.BlockSpec((BLOCK_ROWS, IMW), lambda i: (i, 0)),
        out_shape=jax.ShapeDtypeStruct((ROWS, IMW), input.dtype),
    )(x2, perm)
    return out.reshape(input.shape)


# speedup vs baseline: 16.7507x; 1.1820x over previous
import jax
import jax.numpy as jnp
from jax.experimental import pallas as pl

IMW = 224
ROWS = 8 * 192 * 224
BLOCK_ROWS = 8192


def _flip_body(x_ref, p_ref, o_ref):
    o_ref[...] = jnp.dot(x_ref[...], p_ref[...],
                         preferred_element_type=jnp.float32)


def kernel(input, inv_indices):
    x2 = input.reshape(ROWS, IMW)
    perm = (inv_indices[None, :].astype(jnp.int32)
            == jnp.arange(IMW, dtype=jnp.int32)[:, None]).astype(jnp.float32)
    out = pl.pallas_call(
        _flip_body,
        grid=(ROWS // BLOCK_ROWS,),
        in_specs=[
            pl.BlockSpec((BLOCK_ROWS, IMW), lambda i: (i, 0)),
            pl.BlockSpec((IMW, IMW), lambda i: (0, 0)),
        ],
        out_specs=pl.BlockSpec((BLOCK_ROWS, IMW), lambda i: (i, 0)),
        out_shape=jax.ShapeDtypeStruct((ROWS, IMW), input.dtype),
    )(x2, perm)
    return out.reshape(input.shape)


# speedup vs baseline: 16.8522x; 1.0061x over previous
import jax
import jax.numpy as jnp
from jax.experimental import pallas as pl

IMW = 224
ROWS = 8 * 192 * 224
BLOCK_ROWS = 12288


def _flip_body(x_ref, p_ref, o_ref):
    o_ref[...] = jnp.dot(x_ref[...], p_ref[...],
                         preferred_element_type=jnp.float32)


def kernel(input, inv_indices):
    x2 = input.reshape(ROWS, IMW)
    perm = (inv_indices[None, :].astype(jnp.int32)
            == jnp.arange(IMW, dtype=jnp.int32)[:, None]).astype(jnp.float32)
    out = pl.pallas_call(
        _flip_body,
        grid=(ROWS // BLOCK_ROWS,),
        in_specs=[
            pl.BlockSpec((BLOCK_ROWS, IMW), lambda i: (i, 0)),
            pl.BlockSpec((IMW, IMW), lambda i: (0, 0)),
        ],
        out_specs=pl.BlockSpec((BLOCK_ROWS, IMW), lambda i: (i, 0)),
        out_shape=jax.ShapeDtypeStruct((ROWS, IMW), input.dtype),
    )(x2, perm)
    return out.reshape(input.shape)


# speedup vs baseline: 16.8895x; 1.0022x over previous
import jax
import jax.numpy as jnp
from jax.experimental import pallas as pl

IMW = 224
ROWS = 8 * 192 * 224
BLOCK_ROWS = 14336


def _flip_body(x_ref, p_ref, o_ref):
    o_ref[...] = jnp.dot(x_ref[...], p_ref[...],
                         preferred_element_type=jnp.float32)


def kernel(input, inv_indices):
    x2 = input.reshape(ROWS, IMW)
    perm = (inv_indices[None, :].astype(jnp.int32)
            == jnp.arange(IMW, dtype=jnp.int32)[:, None]).astype(jnp.float32)
    out = pl.pallas_call(
        _flip_body,
        grid=(ROWS // BLOCK_ROWS,),
        in_specs=[
            pl.BlockSpec((BLOCK_ROWS, IMW), lambda i: (i, 0)),
            pl.BlockSpec((IMW, IMW), lambda i: (0, 0)),
        ],
        out_specs=pl.BlockSpec((BLOCK_ROWS, IMW), lambda i: (i, 0)),
        out_shape=jax.ShapeDtypeStruct((ROWS, IMW), input.dtype),
    )(x2, perm)
    return out.reshape(input.shape)
